# baseline (speedup 1.0000x reference)
import functools

import jax
import jax.numpy as jnp
from jax import lax
from jax.experimental import pallas as pl
from jax.experimental.pallas import tpu as pltpu
from jax.experimental.pallas import tpu_sc as plsc

N_VOCAB = 100000
EMBED_DIM = 64
BATCH = 4096
HIST = 50

NC = 2
NS = 16
NW = NC * NS
BT = BATCH // NW

_mesh = plsc.VectorSubcoreMesh(core_axis_name="c", subcore_axis_name="s")


@functools.partial(
    pl.kernel,
    mesh=_mesh,
    out_type=jax.ShapeDtypeStruct((HIST, 8, NW, 8, 128), jnp.float32),
    scratch_types=[
        pltpu.VMEM((HIST, BT), jnp.int32),
        pltpu.VMEM((2, BT, EMBED_DIM), jnp.float32),
        pltpu.VMEM((2, 8, 8, 128), jnp.float32),
        [pltpu.SemaphoreType.DMA] * 2,
        [pltpu.SemaphoreType.DMA] * 2,
    ],
    compiler_params=pltpu.CompilerParams(
        use_tc_tiling_on_sc=False, needs_layout_passes=False),
)
def _embed_lookup(xt_hbm, table_hbm, out_hbm, idx_v, rows_v, tile_v,
                  gsems, ssems):
    wid = lax.axis_index("s") * NC + lax.axis_index("c")
    pltpu.sync_copy(xt_hbm.at[:, pl.ds(wid * BT, BT)], idx_v)

    iotas = [lax.iota(jnp.int32, 16) + 16 * j for j in range(8)]

    def fire(h, b):
        pltpu.async_copy(table_hbm.at[idx_v.at[h]], rows_v.at[b], gsems[b])

    def wait_gather(h, b):
        pltpu.make_async_copy(
            table_hbm.at[idx_v.at[h]], rows_v.at[b], gsems[b]).wait()

    def transpose(b):
        for te in range(8):
            for e_ in range(8):
                col = jnp.full((16,), te * 8 + e_, jnp.int32)
                for j in range(8):
                    vals = plsc.load_gather(rows_v.at[b], [iotas[j], col])
                    tile_v[b, te, e_, pl.ds(16 * j, 16)] = vals

    def start_store(h, b):
        pltpu.async_copy(tile_v.at[b], out_hbm.at[h, :, wid], ssems[b])

    def wait_store(h, b):
        pltpu.make_async_copy(
            tile_v.at[b], out_hbm.at[h, :, wid], ssems[b]).wait()

    fire(0, 0)
    fire(1, 1)

    def body(p, carry):
        for b in (0, 1):
            h = 2 * p + b
            wait_gather(h, b)

            @pl.when(p >= 1)
            def _():
                wait_store(h, b)

            transpose(b)
            start_store(h, b)
            fire(h + 2, b)
        return carry

    lax.fori_loop(0, HIST // 2 - 1, body, 0)

    for b in (0, 1):
        h = HIST - 2 + b
        wait_gather(h, b)
        wait_store(h, b)
        transpose(b)
        start_store(h, b)
    for b in (0, 1):
        wait_store(HIST - 2 + b, b)


def kernel(x, weight):
    xt = x.T.astype(jnp.int32)
    out5 = _embed_lookup(xt, weight)
    return out5.transpose(2, 4, 0, 1, 3).reshape(BATCH, HIST, EMBED_DIM)


# speedup vs baseline: 1.7383x; 1.7383x over previous
import functools

import jax
import jax.numpy as jnp
from jax import lax
from jax.experimental import pallas as pl
from jax.experimental.pallas import tpu as pltpu
from jax.experimental.pallas import tpu_sc as plsc

N_VOCAB = 100000
EMBED_DIM = 64
BATCH = 4096
HIST = 50

NC = 2
NS = 16
NW = NC * NS
BT = BATCH // NW

_mesh = plsc.VectorSubcoreMesh(core_axis_name="c", subcore_axis_name="s")


@functools.partial(
    pl.kernel,
    mesh=_mesh,
    out_type=jax.ShapeDtypeStruct((HIST, 8, NW, 8, 128), jnp.float32),
    scratch_types=[
        pltpu.VMEM((HIST, BT), jnp.int32),
        pltpu.VMEM((2, BT, EMBED_DIM), jnp.float32),
        pltpu.VMEM((2, 8, 8, 128), jnp.float32),
        [pltpu.SemaphoreType.DMA] * 2,
        [pltpu.SemaphoreType.DMA] * 2,
    ],
    compiler_params=pltpu.CompilerParams(
        use_tc_tiling_on_sc=False, needs_layout_passes=False),
)
def _embed_lookup(xt_hbm, table_hbm, out_hbm, idx_v, rows_v, tile_v,
                  gsems, ssems):
    wid = lax.axis_index("s") * NC + lax.axis_index("c")
    pltpu.sync_copy(xt_hbm.at[:, pl.ds(wid * BT, BT)], idx_v)

    iotas = [lax.iota(jnp.int32, 16) + 16 * j for j in range(8)]

    def fire(h, b):
        pltpu.async_copy(table_hbm.at[idx_v.at[h]], rows_v.at[b], gsems[b])

    def wait_gather(h, b):
        pltpu.make_async_copy(
            table_hbm.at[idx_v.at[h]], rows_v.at[b], gsems[b]).wait()

    def transpose(b):
        @plsc.parallel_loop(0, EMBED_DIM, step=1, unroll=8)
        def _(e):
            te = e >> 3
            e_ = e & 7
            col = jnp.full((16,), 0, jnp.int32) + e
            for j in range(8):
                vals = plsc.load_gather(rows_v.at[b], [iotas[j], col])
                tile_v[b, te, e_, pl.ds(16 * j, 16)] = vals

    def start_store(h, b):
        pltpu.async_copy(tile_v.at[b], out_hbm.at[h, :, wid], ssems[b])

    def wait_store(h, b):
        pltpu.make_async_copy(
            tile_v.at[b], out_hbm.at[h, :, wid], ssems[b]).wait()

    fire(0, 0)
    fire(1, 1)

    def body(p, carry):
        for b in (0, 1):
            h = 2 * p + b
            wait_gather(h, b)

            @pl.when(p >= 1)
            def _():
                wait_store(h, b)

            transpose(b)
            start_store(h, b)
            fire(h + 2, b)
        return carry

    lax.fori_loop(0, HIST // 2 - 1, body, 0)

    for b in (0, 1):
        h = HIST - 2 + b
        wait_gather(h, b)
        wait_store(h, b)
        transpose(b)
        start_store(h, b)
    for b in (0, 1):
        wait_store(HIST - 2 + b, b)


def kernel(x, weight):
    xt = x.T.astype(jnp.int32)
    out5 = _embed_lookup(xt, weight)
    return out5.transpose(2, 4, 0, 1, 3).reshape(BATCH, HIST, EMBED_DIM)


# speedup vs baseline: 3.4152x; 1.9647x over previous
import functools

import jax
import jax.numpy as jnp
from jax import lax
from jax.experimental import pallas as pl
from jax.experimental.pallas import tpu as pltpu
from jax.experimental.pallas import tpu_sc as plsc

N_VOCAB = 100000
EMBED_DIM = 64
BATCH = 4096
HIST = 50

NC = 2
NS = 16
NW = NC * NS
BT = BATCH // NW

_mesh = plsc.VectorSubcoreMesh(core_axis_name="c", subcore_axis_name="s")


@functools.partial(
    pl.kernel,
    mesh=_mesh,
    out_type=jax.ShapeDtypeStruct((HIST, 8, NW, 8, 128), jnp.float32),
    scratch_types=[
        pltpu.VMEM((HIST, BT), jnp.int32),
        pltpu.VMEM((2, BT, EMBED_DIM), jnp.float32),
        pltpu.VMEM((2, 8, 8, 133), jnp.float32),
        [pltpu.SemaphoreType.DMA] * 2,
        [pltpu.SemaphoreType.DMA] * 2,
    ],
    compiler_params=pltpu.CompilerParams(
        use_tc_tiling_on_sc=False, needs_layout_passes=False),
)
def _embed_lookup(xt_hbm, table_hbm, out_hbm, idx_v, rows_v, tile_v,
                  gsems, ssems):
    wid = lax.axis_index("s") * NC + lax.axis_index("c")
    pltpu.sync_copy(xt_hbm.at[:, pl.ds(wid * BT, BT)], idx_v)

    iota16 = lax.iota(jnp.int32, 16)
    tevs = [(iota16 + 16 * c) >> 3 for c in range(4)]
    eevs = [(iota16 + 16 * c) & 7 for c in range(4)]

    def fire(h, b):
        pltpu.async_copy(table_hbm.at[idx_v.at[h]], rows_v.at[b], gsems[b])

    def wait_gather(h, b):
        pltpu.make_async_copy(
            table_hbm.at[idx_v.at[h]], rows_v.at[b], gsems[b]).wait()

    def transpose(b):
        @plsc.parallel_loop(0, BT, step=1, unroll=4)
        def _(bp):
            colb = jnp.full((16,), 0, jnp.int32) + bp
            for c in range(4):
                vals = rows_v[b, bp, pl.ds(16 * c, 16)]
                plsc.store_scatter(
                    tile_v.at[b], [tevs[c], eevs[c], colb], vals)

    def start_store(h, b):
        pltpu.async_copy(
            tile_v.at[b, :, :, pl.ds(0, 128)], out_hbm.at[h, :, wid],
            ssems[b])

    def wait_store(h, b):
        pltpu.make_async_copy(
            tile_v.at[b, :, :, pl.ds(0, 128)], out_hbm.at[h, :, wid],
            ssems[b]).wait()

    fire(0, 0)
    fire(1, 1)

    def body(p, carry):
        for b in (0, 1):
            h = 2 * p + b
            wait_gather(h, b)

            @pl.when(p >= 1)
            def _():
                wait_store(h, b)

            transpose(b)
            start_store(h, b)
            fire(h + 2, b)
        return carry

    lax.fori_loop(0, HIST // 2 - 1, body, 0)

    for b in (0, 1):
        h = HIST - 2 + b
        wait_gather(h, b)
        wait_store(h, b)
        transpose(b)
        start_store(h, b)
    for b in (0, 1):
        wait_store(HIST - 2 + b, b)


def kernel(x, weight):
    xt = x.T.astype(jnp.int32)
    out5 = _embed_lookup(xt, weight)
    return out5.transpose(2, 4, 0, 1, 3).reshape(BATCH, HIST, EMBED_DIM)
